# baseline (speedup 1.0000x reference)
import functools

import jax
import jax.numpy as jnp
import numpy as np
from jax import lax
from jax.experimental import pallas as pl
from jax.experimental.pallas import tpu as pltpu
from jax.experimental.pallas import tpu_sc as plsc

OUT_SZ = 5
G = OUT_SZ ** 3
GP = 128
R = 4
NW = 32
RPW = 4


def _base_grid_t():
    b = np.arange(0, 1, 1.0 / OUT_SZ) - (OUT_SZ - 1) / (2.0 * OUT_SZ)
    gx = np.tile(b.reshape(-1, 1, 1), (1, OUT_SZ, OUT_SZ))
    gy = np.tile(b.reshape(1, -1, 1), (OUT_SZ, 1, OUT_SZ))
    gz = np.tile(b.reshape(1, 1, -1), (OUT_SZ, OUT_SZ, 1))
    g = np.stack([gx, gy, gz], axis=-1).reshape(-1, 3).T
    g = np.concatenate([g, np.tile(g[:, -1:], (1, GP - G))], axis=1)
    return jnp.asarray(g, dtype=jnp.float32)


def _nn_kernel(bg_ref, rois_ref, pts_ref, idx_ref, wts_ref, *, S):
    bg = bg_ref[...]
    pts = pts_ref[0]
    normp = jnp.sum(pts * pts, axis=1, keepdims=True)
    sidx = lax.broadcasted_iota(jnp.int32, (S, GP), 0)
    boff = pl.program_id(0) * S
    for r in range(R):
        row = rois_ref[0, r]
        ctr_z = row[0:1, 2:3] + 0.5 * row[0:1, 5:6]
        qx = bg[0:1, :] * row[0:1, 3:4] + row[0:1, 0:1]
        qy = bg[1:2, :] * row[0:1, 4:5] + row[0:1, 1:2]
        qz = bg[2:3, :] * row[0:1, 5:6] + ctr_z
        q = jnp.concatenate([qx, qy, qz], axis=0)
        cross = jnp.dot(pts, q, preferred_element_type=jnp.float32)
        normq = jnp.sum(q * q, axis=0, keepdims=True)
        d2 = normq - 2.0 * cross + normp
        dwork = d2
        recips, idxs = [], []
        for k in range(3):
            m = jnp.min(dwork, axis=0, keepdims=True)
            eq = dwork == m
            ik = jnp.min(jnp.where(eq, sidx, S), axis=0, keepdims=True)
            recips.append(1.0 / (jnp.sqrt(jnp.maximum(m, 0.0)) + 1e-8))
            idxs.append(ik)
            if k < 2:
                dwork = jnp.where(sidx == ik, jnp.inf, dwork)
        norm = recips[0] + recips[1] + recips[2]
        idx_ref[0, r] = jnp.concatenate(idxs, axis=0) + boff
        wts_ref[0, r] = jnp.concatenate(
            [recips[0] / norm, recips[1] / norm, recips[2] / norm], axis=0)


def _sc_gather_kernel(idx_hbm, table_hbm, out_hbm, idx_v, rows_v, sem):
    wid = lax.axis_index("s") * 2 + lax.axis_index("c")
    for r in range(RPW):
        roi = wid * RPW + r
        pltpu.sync_copy(idx_hbm.at[roi], idx_v)
        cps = [pltpu.async_copy(table_hbm.at[idx_v.at[k]], rows_v.at[k], sem)
               for k in range(3)]
        for cp in cps:
            cp.wait()
        pltpu.sync_copy(rows_v, out_hbm.at[roi])


def _mlp_kernel(i_ref, w_ref, wt_ref, h_ref, sum_ref, sq_ref):
    rows = i_ref[0]
    w = w_ref[0]
    it = (w[:, 0:1] * rows[0] + w[:, 1:2] * rows[1]
          + w[:, 2:3] * rows[2])
    qmask = lax.broadcasted_iota(jnp.int32, (GP, 1), 0) < G
    it = jnp.where(qmask, it, 0.0)
    h_q = jnp.dot(it, wt_ref[...], preferred_element_type=jnp.float32)
    h_ref[0] = jnp.transpose(h_q)[:, 0:G]
    sum_ref[0] = jnp.sum(h_q, axis=0, keepdims=True)
    sq_ref[0] = jnp.sum(h_q * h_q, axis=0, keepdims=True)


def _bn_kernel(h_ref, sum_ref, sq_ref, g_ref, b_ref, o_ref, *, count):
    mean = sum_ref[...] / count
    var = sq_ref[...] / count - mean * mean
    scale = g_ref[...] * lax.rsqrt(var + 1e-5)
    shift = b_ref[...] - mean * scale
    o_ref[0] = jnp.maximum(h_ref[0] * scale + shift, 0.0)


def kernel(pts, pts_feature, rois, W, gamma, beta):
    B, S, _ = pts.shape
    C = pts_feature.shape[1]
    N = rois.shape[1]
    NT = N // R
    BN = B * N
    rois4 = rois.reshape(B, N, 1, 7)
    bg = _base_grid_t()

    idx4, wts4 = pl.pallas_call(
        functools.partial(_nn_kernel, S=S),
        grid=(B, NT),
        in_specs=[
            pl.BlockSpec((3, GP), lambda b, n: (0, 0)),
            pl.BlockSpec((1, R, 1, 7), lambda b, n: (b, n, 0, 0)),
            pl.BlockSpec((1, S, 3), lambda b, n: (b, 0, 0)),
        ],
        out_specs=[
            pl.BlockSpec((1, R, 3, GP), lambda b, n: (b, n, 0, 0)),
            pl.BlockSpec((1, R, 3, GP), lambda b, n: (b, n, 0, 0)),
        ],
        out_shape=[
            jax.ShapeDtypeStruct((B, N, 3, GP), jnp.int32),
            jax.ShapeDtypeStruct((B, N, 3, GP), jnp.float32),
        ],
    )(bg, rois4, pts)

    feats_flat = pts_feature.transpose(0, 2, 1).reshape(B * S, C)
    idx_flat = idx4.reshape(BN, 3, GP)
    wq = wts4.reshape(BN, 3, GP).transpose(0, 2, 1)

    sc_gather = pl.kernel(
        _sc_gather_kernel,
        mesh=plsc.VectorSubcoreMesh(core_axis_name="c", subcore_axis_name="s"),
        out_type=jax.ShapeDtypeStruct((BN, 3, GP, C), jnp.float32),
        scratch_types=[
            pltpu.VMEM((3, GP), jnp.int32),
            pltpu.VMEM((3, GP, C), jnp.float32),
            pltpu.SemaphoreType.DMA,
        ],
    )
    rows3 = sc_gather(idx_flat, feats_flat)

    h4, psum, psq = pl.pallas_call(
        _mlp_kernel,
        grid=(BN,),
        in_specs=[
            pl.BlockSpec((1, 3, GP, C), lambda i: (i, 0, 0, 0)),
            pl.BlockSpec((1, GP, 3), lambda i: (i, 0, 0)),
            pl.BlockSpec((C, C), lambda i: (0, 0)),
        ],
        out_specs=[
            pl.BlockSpec((1, C, G), lambda i: (i, 0, 0)),
            pl.BlockSpec((1, 1, C), lambda i: (i, 0, 0)),
            pl.BlockSpec((1, 1, C), lambda i: (i, 0, 0)),
        ],
        out_shape=[
            jax.ShapeDtypeStruct((BN, C, G), jnp.float32),
            jax.ShapeDtypeStruct((BN, 1, C), jnp.float32),
            jax.ShapeDtypeStruct((BN, 1, C), jnp.float32),
        ],
    )(rows3, wq, W.T)

    ssum = jnp.sum(psum, axis=0).reshape(C, 1)
    ssq = jnp.sum(psq, axis=0).reshape(C, 1)

    out = pl.pallas_call(
        functools.partial(_bn_kernel, count=float(B * N * G)),
        grid=(BN,),
        in_specs=[
            pl.BlockSpec((1, C, G), lambda i: (i, 0, 0)),
            pl.BlockSpec((C, 1), lambda i: (0, 0)),
            pl.BlockSpec((C, 1), lambda i: (0, 0)),
            pl.BlockSpec((C, 1), lambda i: (0, 0)),
            pl.BlockSpec((C, 1), lambda i: (0, 0)),
        ],
        out_specs=pl.BlockSpec((1, C, G), lambda i: (i, 0, 0)),
        out_shape=jax.ShapeDtypeStruct((BN, C, G), jnp.float32),
    )(h4, ssum, ssq, gamma.reshape(C, 1), beta.reshape(C, 1))
    return out


# speedup vs baseline: 1.3254x; 1.3254x over previous
import functools

import jax
import jax.numpy as jnp
import numpy as np
from jax.experimental import pallas as pl
from jax.experimental.pallas import tpu as pltpu

OUT_SZ = 5
G = OUT_SZ ** 3
R = 4


def _base_grid_t():
    b = np.arange(0, 1, 1.0 / OUT_SZ) - (OUT_SZ - 1) / (2.0 * OUT_SZ)
    gx = np.tile(b.reshape(-1, 1, 1), (1, OUT_SZ, OUT_SZ))
    gy = np.tile(b.reshape(1, -1, 1), (OUT_SZ, 1, OUT_SZ))
    gz = np.tile(b.reshape(1, 1, -1), (OUT_SZ, OUT_SZ, 1))
    g = np.stack([gx, gy, gz], axis=-1).reshape(-1, 3)
    return jnp.asarray(g.T, dtype=jnp.float32)


def _fp_kernel(bg_ref, rois_ref, pts_ref, feats_ref, w_ref,
               h_ref, sum_ref, sq_ref):
    feats = feats_ref[0]
    bg = bg_ref[...]
    Wm = w_ref[...]
    pts = pts_ref[0]
    normp = jnp.sum(pts * pts, axis=1, keepdims=True)
    acc_s = jnp.zeros((Wm.shape[0], 1), jnp.float32)
    acc_q = jnp.zeros((Wm.shape[0], 1), jnp.float32)
    for r in range(R):
        row = rois_ref[0, r]
        ctr_z = row[0:1, 2:3] + 0.5 * row[0:1, 5:6]
        qx = bg[0:1, :] * row[0:1, 3:4] + row[0:1, 0:1]
        qy = bg[1:2, :] * row[0:1, 4:5] + row[0:1, 1:2]
        qz = bg[2:3, :] * row[0:1, 5:6] + ctr_z
        q = jnp.concatenate([qx, qy, qz], axis=0)
        cross = jnp.dot(pts, q, preferred_element_type=jnp.float32)
        normq = jnp.sum(q * q, axis=0, keepdims=True)
        d2 = normq - 2.0 * cross + normp
        dwork = d2
        recips, eqs = [], []
        for k in range(3):
            m = jnp.min(dwork, axis=0, keepdims=True)
            eq = dwork == m
            recips.append(1.0 / (jnp.sqrt(jnp.maximum(m, 0.0)) + 1e-8))
            eqs.append(eq)
            if k < 2:
                dwork = jnp.where(eq, jnp.inf, dwork)
        norm = recips[0] + recips[1] + recips[2]
        a = jnp.where(eqs[0], recips[0] / norm,
                      jnp.where(eqs[1], recips[1] / norm,
                                jnp.where(eqs[2], recips[2] / norm, 0.0)))
        interp = jnp.dot(feats, a, preferred_element_type=jnp.float32)
        h = jnp.dot(Wm, interp, preferred_element_type=jnp.float32)
        h_ref[0, r] = h
        acc_s = acc_s + jnp.sum(h, axis=1, keepdims=True)
        acc_q = acc_q + jnp.sum(h * h, axis=1, keepdims=True)
    sum_ref[0, 0] = acc_s
    sq_ref[0, 0] = acc_q


def _bn_kernel(h_ref, sum_ref, sq_ref, g_ref, b_ref, o_ref, *, count):
    s = sum_ref[...]
    qq = sq_ref[...]
    mean = s / count
    var = qq / count - mean * mean
    scale = g_ref[...] * jax.lax.rsqrt(var + 1e-5)
    shift = b_ref[...] - mean * scale
    h = h_ref[0, 0]
    o_ref[0] = jnp.maximum(h * scale + shift, 0.0)


def kernel(pts, pts_feature, rois, W, gamma, beta):
    B, S, _ = pts.shape
    C = pts_feature.shape[1]
    N = rois.shape[1]
    NT = N // R
    rois4 = rois.reshape(B, N, 1, 7)
    bg = _base_grid_t()

    h4, psum, psq = pl.pallas_call(
        _fp_kernel,
        grid=(B, NT),
        in_specs=[
            pl.BlockSpec((3, G), lambda b, n: (0, 0)),
            pl.BlockSpec((1, R, 1, 7), lambda b, n: (b, n, 0, 0)),
            pl.BlockSpec((1, S, 3), lambda b, n: (b, 0, 0)),
            pl.BlockSpec((1, C, S), lambda b, n: (b, 0, 0)),
            pl.BlockSpec((C, C), lambda b, n: (0, 0)),
        ],
        out_specs=[
            pl.BlockSpec((1, R, C, G), lambda b, n: (b, n, 0, 0)),
            pl.BlockSpec((1, 1, C, 1), lambda b, n: (b, n, 0, 0)),
            pl.BlockSpec((1, 1, C, 1), lambda b, n: (b, n, 0, 0)),
        ],
        out_shape=[
            jax.ShapeDtypeStruct((B, N, C, G), jnp.float32),
            jax.ShapeDtypeStruct((B, NT, C, 1), jnp.float32),
            jax.ShapeDtypeStruct((B, NT, C, 1), jnp.float32),
        ],
    )(bg, rois4, pts, pts_feature, W)

    ssum = jnp.sum(psum, axis=(0, 1))
    ssq = jnp.sum(psq, axis=(0, 1))

    out = pl.pallas_call(
        functools.partial(_bn_kernel, count=float(B * N * G)),
        grid=(B, N),
        in_specs=[
            pl.BlockSpec((1, 1, C, G), lambda b, n: (b, n, 0, 0)),
            pl.BlockSpec((C, 1), lambda b, n: (0, 0)),
            pl.BlockSpec((C, 1), lambda b, n: (0, 0)),
            pl.BlockSpec((C, 1), lambda b, n: (0, 0)),
            pl.BlockSpec((C, 1), lambda b, n: (0, 0)),
        ],
        out_specs=pl.BlockSpec((1, C, G), lambda b, n, N=N: (b * N + n, 0, 0)),
        out_shape=jax.ShapeDtypeStruct((B * N, C, G), jnp.float32),
    )(h4, ssum, ssq, gamma.reshape(C, 1), beta.reshape(C, 1))
    return out


# speedup vs baseline: 1.6713x; 1.2610x over previous
import functools

import jax
import jax.numpy as jnp
import numpy as np
from jax.experimental import pallas as pl
from jax.experimental.pallas import tpu as pltpu

OUT_SZ = 5
G = OUT_SZ ** 3
R = 4


def _base_grid_t():
    b = np.arange(0, 1, 1.0 / OUT_SZ) - (OUT_SZ - 1) / (2.0 * OUT_SZ)
    gx = np.tile(b.reshape(-1, 1, 1), (1, OUT_SZ, OUT_SZ))
    gy = np.tile(b.reshape(1, -1, 1), (OUT_SZ, 1, OUT_SZ))
    gz = np.tile(b.reshape(1, 1, -1), (OUT_SZ, OUT_SZ, 1))
    g = np.stack([gx, gy, gz], axis=-1).reshape(-1, 3)
    return jnp.asarray(g.T, dtype=jnp.float32)


def _fp_kernel(bg_ref, rois_ref, pts_ref, feats_ref, w_ref,
               h_ref, sum_ref, sq_ref):
    feats = feats_ref[0]
    bg = bg_ref[...]
    Wm = w_ref[...]
    pts = pts_ref[0]
    S = pts.shape[0]
    normp = jnp.sum(pts * pts, axis=1, keepdims=True)
    acc_s = jnp.zeros((Wm.shape[0], 1), jnp.float32)
    acc_q = jnp.zeros((Wm.shape[0], 1), jnp.float32)
    for r in range(R):
        row = rois_ref[0, r]
        ctr_z = row[0:1, 2:3] + 0.5 * row[0:1, 5:6]
        qx = bg[0:1, :] * row[0:1, 3:4] + row[0:1, 0:1]
        qy = bg[1:2, :] * row[0:1, 4:5] + row[0:1, 1:2]
        qz = bg[2:3, :] * row[0:1, 5:6] + ctr_z
        q = jnp.concatenate([qx, qy, qz], axis=0)
        cross = jnp.dot(pts, q, preferred_element_type=jnp.float32)
        normq = jnp.sum(q * q, axis=0, keepdims=True)
        d2 = normq - 2.0 * cross + normp
        CH = 128
        m1 = d2[0:CH]
        m2 = jnp.full((CH, G), jnp.inf, jnp.float32)
        m3 = m2
        for j in range(1, S // CH):
            v = d2[j * CH:(j + 1) * CH]
            t1 = jnp.maximum(m1, v)
            m1 = jnp.minimum(m1, v)
            t2 = jnp.maximum(m2, t1)
            m2 = jnp.minimum(m2, t1)
            m3 = jnp.minimum(m3, t2)
        cand = jnp.concatenate([m1, m2, m3], axis=0)
        mms, recips = [], []
        for k in range(3):
            mm = jnp.min(cand, axis=0, keepdims=True)
            mms.append(mm)
            recips.append(1.0 / (jnp.sqrt(jnp.maximum(mm, 0.0)) + 1e-8))
            if k < 2:
                cand = jnp.where(cand == mm, jnp.inf, cand)
        norm = recips[0] + recips[1] + recips[2]
        a = jnp.where(d2 == mms[0], recips[0] / norm,
                      jnp.where(d2 == mms[1], recips[1] / norm,
                                jnp.where(d2 == mms[2], recips[2] / norm, 0.0)))
        interp = jnp.dot(feats, a, preferred_element_type=jnp.float32)
        h = jnp.dot(Wm, interp, preferred_element_type=jnp.float32)
        h_ref[0, r] = h
        acc_s = acc_s + jnp.sum(h, axis=1, keepdims=True)
        acc_q = acc_q + jnp.sum(h * h, axis=1, keepdims=True)
    sum_ref[0, 0] = acc_s
    sq_ref[0, 0] = acc_q


def _bn_kernel(h_ref, sum_ref, sq_ref, g_ref, b_ref, o_ref, *, count):
    s = sum_ref[...]
    qq = sq_ref[...]
    mean = s / count
    var = qq / count - mean * mean
    scale = g_ref[...] * jax.lax.rsqrt(var + 1e-5)
    shift = b_ref[...] - mean * scale
    h = h_ref[0, 0]
    o_ref[0] = jnp.maximum(h * scale + shift, 0.0)


def kernel(pts, pts_feature, rois, W, gamma, beta):
    B, S, _ = pts.shape
    C = pts_feature.shape[1]
    N = rois.shape[1]
    NT = N // R
    rois4 = rois.reshape(B, N, 1, 7)
    bg = _base_grid_t()

    h4, psum, psq = pl.pallas_call(
        _fp_kernel,
        grid=(B, NT),
        in_specs=[
            pl.BlockSpec((3, G), lambda b, n: (0, 0)),
            pl.BlockSpec((1, R, 1, 7), lambda b, n: (b, n, 0, 0)),
            pl.BlockSpec((1, S, 3), lambda b, n: (b, 0, 0)),
            pl.BlockSpec((1, C, S), lambda b, n: (b, 0, 0)),
            pl.BlockSpec((C, C), lambda b, n: (0, 0)),
        ],
        out_specs=[
            pl.BlockSpec((1, R, C, G), lambda b, n: (b, n, 0, 0)),
            pl.BlockSpec((1, 1, C, 1), lambda b, n: (b, n, 0, 0)),
            pl.BlockSpec((1, 1, C, 1), lambda b, n: (b, n, 0, 0)),
        ],
        out_shape=[
            jax.ShapeDtypeStruct((B, N, C, G), jnp.float32),
            jax.ShapeDtypeStruct((B, NT, C, 1), jnp.float32),
            jax.ShapeDtypeStruct((B, NT, C, 1), jnp.float32),
        ],
    )(bg, rois4, pts, pts_feature, W)

    ssum = jnp.sum(psum, axis=(0, 1))
    ssq = jnp.sum(psq, axis=(0, 1))

    out = pl.pallas_call(
        functools.partial(_bn_kernel, count=float(B * N * G)),
        grid=(B, N),
        in_specs=[
            pl.BlockSpec((1, 1, C, G), lambda b, n: (b, n, 0, 0)),
            pl.BlockSpec((C, 1), lambda b, n: (0, 0)),
            pl.BlockSpec((C, 1), lambda b, n: (0, 0)),
            pl.BlockSpec((C, 1), lambda b, n: (0, 0)),
            pl.BlockSpec((C, 1), lambda b, n: (0, 0)),
        ],
        out_specs=pl.BlockSpec((1, C, G), lambda b, n, N=N: (b * N + n, 0, 0)),
        out_shape=jax.ShapeDtypeStruct((B * N, C, G), jnp.float32),
    )(h4, ssum, ssq, gamma.reshape(C, 1), beta.reshape(C, 1))
    return out


# speedup vs baseline: 2.1442x; 1.2829x over previous
import functools

import jax
import jax.numpy as jnp
import numpy as np
from jax.experimental import pallas as pl
from jax.experimental.pallas import tpu as pltpu

OUT_SZ = 5
G = OUT_SZ ** 3
R = 8
RB = 8


def _base_grid_t():
    b = np.arange(0, 1, 1.0 / OUT_SZ) - (OUT_SZ - 1) / (2.0 * OUT_SZ)
    gx = np.tile(b.reshape(-1, 1, 1), (1, OUT_SZ, OUT_SZ))
    gy = np.tile(b.reshape(1, -1, 1), (OUT_SZ, 1, OUT_SZ))
    gz = np.tile(b.reshape(1, 1, -1), (OUT_SZ, OUT_SZ, 1))
    g = np.stack([gx, gy, gz], axis=-1).reshape(-1, 3)
    return jnp.asarray(g.T, dtype=jnp.float32)


def _fp_kernel(bg_ref, rois_ref, pts_ref, feats_ref, w_ref,
               h_ref, sum_ref, sq_ref):
    feats = feats_ref[0]
    bg = bg_ref[...]
    Wm = w_ref[...]
    pts = pts_ref[0]
    S = pts.shape[0]
    normp = jnp.sum(pts * pts, axis=1, keepdims=True)
    acc_s = jnp.zeros((Wm.shape[0], 1), jnp.float32)
    acc_q = jnp.zeros((Wm.shape[0], 1), jnp.float32)
    for r in range(R):
        row = rois_ref[0, r]
        ctr_z = row[0:1, 2:3] + 0.5 * row[0:1, 5:6]
        qx = bg[0:1, :] * row[0:1, 3:4] + row[0:1, 0:1]
        qy = bg[1:2, :] * row[0:1, 4:5] + row[0:1, 1:2]
        qz = bg[2:3, :] * row[0:1, 5:6] + ctr_z
        q = jnp.concatenate([qx, qy, qz], axis=0)
        cross = jnp.dot(pts, q, preferred_element_type=jnp.float32)
        normq = jnp.sum(q * q, axis=0, keepdims=True)
        d2 = normq - 2.0 * cross + normp
        CH = 128
        m1 = d2[0:CH]
        m2 = jnp.full((CH, G), jnp.inf, jnp.float32)
        m3 = m2
        for j in range(1, S // CH):
            v = d2[j * CH:(j + 1) * CH]
            t1 = jnp.maximum(m1, v)
            m1 = jnp.minimum(m1, v)
            t2 = jnp.maximum(m2, t1)
            m2 = jnp.minimum(m2, t1)
            m3 = jnp.minimum(m3, t2)
        cand = jnp.concatenate([m1, m2, m3], axis=0)
        mms, recips = [], []
        for k in range(3):
            mm = jnp.min(cand, axis=0, keepdims=True)
            mms.append(mm)
            recips.append(1.0 / (jnp.sqrt(jnp.maximum(mm, 0.0)) + 1e-8))
            if k < 2:
                cand = jnp.where(cand == mm, jnp.inf, cand)
        norm = recips[0] + recips[1] + recips[2]
        a = jnp.where(d2 == mms[0], recips[0] / norm,
                      jnp.where(d2 == mms[1], recips[1] / norm,
                                jnp.where(d2 == mms[2], recips[2] / norm, 0.0)))
        interp = jnp.dot(feats, a, preferred_element_type=jnp.float32)
        h = jnp.dot(Wm, interp, preferred_element_type=jnp.float32)
        h_ref[0, r] = h
        acc_s = acc_s + jnp.sum(h, axis=1, keepdims=True)
        acc_q = acc_q + jnp.sum(h * h, axis=1, keepdims=True)
    sum_ref[0, 0] = acc_s
    sq_ref[0, 0] = acc_q


def _bn_kernel(h_ref, sum_ref, sq_ref, g_ref, b_ref, o_ref, *, count):
    s = sum_ref[...]
    qq = sq_ref[...]
    mean = s / count
    var = qq / count - mean * mean
    scale = g_ref[...] * jax.lax.rsqrt(var + 1e-5)
    shift = b_ref[...] - mean * scale
    for r in range(RB):
        o_ref[r] = jnp.maximum(h_ref[0, r] * scale + shift, 0.0)


def kernel(pts, pts_feature, rois, W, gamma, beta):
    B, S, _ = pts.shape
    C = pts_feature.shape[1]
    N = rois.shape[1]
    NT = N // R
    rois4 = rois.reshape(B, N, 1, 7)
    bg = _base_grid_t()

    h4, psum, psq = pl.pallas_call(
        _fp_kernel,
        grid=(B, NT),
        in_specs=[
            pl.BlockSpec((3, G), lambda b, n: (0, 0)),
            pl.BlockSpec((1, R, 1, 7), lambda b, n: (b, n, 0, 0)),
            pl.BlockSpec((1, S, 3), lambda b, n: (b, 0, 0)),
            pl.BlockSpec((1, C, S), lambda b, n: (b, 0, 0)),
            pl.BlockSpec((C, C), lambda b, n: (0, 0)),
        ],
        out_specs=[
            pl.BlockSpec((1, R, C, G), lambda b, n: (b, n, 0, 0)),
            pl.BlockSpec((1, 1, C, 1), lambda b, n: (b, n, 0, 0)),
            pl.BlockSpec((1, 1, C, 1), lambda b, n: (b, n, 0, 0)),
        ],
        out_shape=[
            jax.ShapeDtypeStruct((B, N, C, G), jnp.float32),
            jax.ShapeDtypeStruct((B, NT, C, 1), jnp.float32),
            jax.ShapeDtypeStruct((B, NT, C, 1), jnp.float32),
        ],
    )(bg, rois4, pts, pts_feature, W)

    ssum = jnp.sum(psum, axis=(0, 1))
    ssq = jnp.sum(psq, axis=(0, 1))

    NB = N // RB
    out = pl.pallas_call(
        functools.partial(_bn_kernel, count=float(B * N * G)),
        grid=(B, NB),
        in_specs=[
            pl.BlockSpec((1, RB, C, G), lambda b, n: (b, n, 0, 0)),
            pl.BlockSpec((C, 1), lambda b, n: (0, 0)),
            pl.BlockSpec((C, 1), lambda b, n: (0, 0)),
            pl.BlockSpec((C, 1), lambda b, n: (0, 0)),
            pl.BlockSpec((C, 1), lambda b, n: (0, 0)),
        ],
        out_specs=pl.BlockSpec((RB, C, G),
                               lambda b, n, NB=NB: (b * NB + n, 0, 0)),
        out_shape=jax.ShapeDtypeStruct((B * N, C, G), jnp.float32),
    )(h4, ssum, ssq, gamma.reshape(C, 1), beta.reshape(C, 1))
    return out


# speedup vs baseline: 2.1787x; 1.0161x over previous
import functools

import jax
import jax.numpy as jnp
import numpy as np
from jax.experimental import pallas as pl
from jax.experimental.pallas import tpu as pltpu

OUT_SZ = 5
G = OUT_SZ ** 3
R = 16
RB = 8


def _base_grid_t():
    b = np.arange(0, 1, 1.0 / OUT_SZ) - (OUT_SZ - 1) / (2.0 * OUT_SZ)
    gx = np.tile(b.reshape(-1, 1, 1), (1, OUT_SZ, OUT_SZ))
    gy = np.tile(b.reshape(1, -1, 1), (OUT_SZ, 1, OUT_SZ))
    gz = np.tile(b.reshape(1, 1, -1), (OUT_SZ, OUT_SZ, 1))
    g = np.stack([gx, gy, gz], axis=-1).reshape(-1, 3)
    return jnp.asarray(g.T, dtype=jnp.float32)


def _fp_kernel(bg_ref, rois_ref, pts_ref, feats_ref, w_ref,
               h_ref, sum_ref, sq_ref):
    feats = feats_ref[0]
    bg = bg_ref[...]
    Wm = w_ref[...]
    pts = pts_ref[0]
    S = pts.shape[0]
    normp = jnp.sum(pts * pts, axis=1, keepdims=True)
    acc_s = jnp.zeros((Wm.shape[0], 1), jnp.float32)
    acc_q = jnp.zeros((Wm.shape[0], 1), jnp.float32)
    for r in range(R):
        row = rois_ref[0, r]
        ctr_z = row[0:1, 2:3] + 0.5 * row[0:1, 5:6]
        qx = bg[0:1, :] * row[0:1, 3:4] + row[0:1, 0:1]
        qy = bg[1:2, :] * row[0:1, 4:5] + row[0:1, 1:2]
        qz = bg[2:3, :] * row[0:1, 5:6] + ctr_z
        q = jnp.concatenate([qx, qy, qz], axis=0)
        cross = jnp.dot(pts, q, preferred_element_type=jnp.float32)
        normq = jnp.sum(q * q, axis=0, keepdims=True)
        d2 = normq - 2.0 * cross + normp
        CH = 128
        m1 = d2[0:CH]
        m2 = jnp.full((CH, G), jnp.inf, jnp.float32)
        m3 = m2
        for j in range(1, S // CH):
            v = d2[j * CH:(j + 1) * CH]
            t1 = jnp.maximum(m1, v)
            m1 = jnp.minimum(m1, v)
            t2 = jnp.maximum(m2, t1)
            m2 = jnp.minimum(m2, t1)
            m3 = jnp.minimum(m3, t2)
        cand = jnp.concatenate([m1, m2, m3], axis=0)
        mms, recips = [], []
        for k in range(3):
            mm = jnp.min(cand, axis=0, keepdims=True)
            mms.append(mm)
            recips.append(1.0 / (jnp.sqrt(jnp.maximum(mm, 0.0)) + 1e-8))
            if k < 2:
                cand = jnp.where(cand == mm, jnp.inf, cand)
        norm = recips[0] + recips[1] + recips[2]
        a = jnp.where(d2 == mms[0], recips[0] / norm,
                      jnp.where(d2 == mms[1], recips[1] / norm,
                                jnp.where(d2 == mms[2], recips[2] / norm, 0.0)))
        interp = jnp.dot(feats, a, preferred_element_type=jnp.float32)
        h = jnp.dot(Wm, interp, preferred_element_type=jnp.float32)
        h_ref[0, r] = h
        acc_s = acc_s + jnp.sum(h, axis=1, keepdims=True)
        acc_q = acc_q + jnp.sum(h * h, axis=1, keepdims=True)
    sum_ref[0, 0] = acc_s
    sq_ref[0, 0] = acc_q


def _bn_kernel(h_ref, sum_ref, sq_ref, g_ref, b_ref, o_ref, *, count):
    s = sum_ref[...]
    qq = sq_ref[...]
    mean = s / count
    var = qq / count - mean * mean
    scale = g_ref[...] * jax.lax.rsqrt(var + 1e-5)
    shift = b_ref[...] - mean * scale
    for r in range(RB):
        o_ref[r] = jnp.maximum(h_ref[0, r] * scale + shift, 0.0)


def kernel(pts, pts_feature, rois, W, gamma, beta):
    B, S, _ = pts.shape
    C = pts_feature.shape[1]
    N = rois.shape[1]
    NT = N // R
    rois4 = rois.reshape(B, N, 1, 7)
    bg = _base_grid_t()

    h4, psum, psq = pl.pallas_call(
        _fp_kernel,
        grid=(B, NT),
        in_specs=[
            pl.BlockSpec((3, G), lambda b, n: (0, 0)),
            pl.BlockSpec((1, R, 1, 7), lambda b, n: (b, n, 0, 0)),
            pl.BlockSpec((1, S, 3), lambda b, n: (b, 0, 0)),
            pl.BlockSpec((1, C, S), lambda b, n: (b, 0, 0)),
            pl.BlockSpec((C, C), lambda b, n: (0, 0)),
        ],
        out_specs=[
            pl.BlockSpec((1, R, C, G), lambda b, n: (b, n, 0, 0)),
            pl.BlockSpec((1, 1, C, 1), lambda b, n: (b, n, 0, 0)),
            pl.BlockSpec((1, 1, C, 1), lambda b, n: (b, n, 0, 0)),
        ],
        out_shape=[
            jax.ShapeDtypeStruct((B, N, C, G), jnp.float32),
            jax.ShapeDtypeStruct((B, NT, C, 1), jnp.float32),
            jax.ShapeDtypeStruct((B, NT, C, 1), jnp.float32),
        ],
    )(bg, rois4, pts, pts_feature, W)

    ssum = jnp.sum(psum, axis=(0, 1))
    ssq = jnp.sum(psq, axis=(0, 1))

    NB = N // RB
    out = pl.pallas_call(
        functools.partial(_bn_kernel, count=float(B * N * G)),
        grid=(B, NB),
        in_specs=[
            pl.BlockSpec((1, RB, C, G), lambda b, n: (b, n, 0, 0)),
            pl.BlockSpec((C, 1), lambda b, n: (0, 0)),
            pl.BlockSpec((C, 1), lambda b, n: (0, 0)),
            pl.BlockSpec((C, 1), lambda b, n: (0, 0)),
            pl.BlockSpec((C, 1), lambda b, n: (0, 0)),
        ],
        out_specs=pl.BlockSpec((RB, C, G),
                               lambda b, n, NB=NB: (b * NB + n, 0, 0)),
        out_shape=jax.ShapeDtypeStruct((B * N, C, G), jnp.float32),
    )(h4, ssum, ssq, gamma.reshape(C, 1), beta.reshape(C, 1))
    return out
